# baseline (speedup 1.0000x reference)
import functools

import jax
import jax.numpy as jnp
from jax import lax
from jax.experimental import pallas as pl
from jax.experimental.pallas import tpu as pltpu
from jax.experimental.pallas import tpu_sc as plsc

NUM_WORKERS = 32
LANES = 128
ROWS_PER_WORKER = 50
CHUNK = 10
N_CHUNKS = ROWS_PER_WORKER // CHUNK
EMB_D = 32


def _gather_body(table_hbm, idx_hbm, out_hbm, idx_v, rows_v, sem):
    wid = lax.axis_index("s") * 2 + lax.axis_index("c")
    pltpu.sync_copy(idx_hbm.at[wid], idx_v)
    for c in range(N_CHUNKS):
        descs = [
            pltpu.async_copy(table_hbm.at[idx_v.at[c * CHUNK + j]], rows_v.at[j], sem)
            for j in range(CHUNK)
        ]
        for dsc in descs:
            dsc.wait()
        pltpu.sync_copy(rows_v, out_hbm.at[wid, pl.ds(c * CHUNK, CHUNK)])


@jax.jit
def _embedding_lookup(idx, embedding_matrix):
    mesh = plsc.VectorSubcoreMesh(core_axis_name="c", subcore_axis_name="s")
    k = functools.partial(
        pl.kernel,
        mesh=mesh,
        out_type=jax.ShapeDtypeStruct(
            (NUM_WORKERS, ROWS_PER_WORKER, LANES, EMB_D), jnp.float32),
        scratch_types=[
            pltpu.VMEM((ROWS_PER_WORKER, LANES), jnp.int32),
            pltpu.VMEM((CHUNK, LANES, EMB_D), jnp.float32),
            pltpu.SemaphoreType.DMA,
        ],
        compiler_params=pltpu.CompilerParams(use_tc_tiling_on_sc=False),
    )(_gather_body)
    return k(embedding_matrix, idx)


def kernel(token_ids, embedding_matrix):
    B, S = token_ids.shape
    idx = token_ids.astype(jnp.int32).reshape(NUM_WORKERS, ROWS_PER_WORKER, LANES)
    out = _embedding_lookup(idx, embedding_matrix)
    return out.reshape(B, S, EMB_D)


# speedup vs baseline: 1.4150x; 1.4150x over previous
import functools

import jax
import jax.numpy as jnp
from jax import lax
from jax.experimental import pallas as pl
from jax.experimental.pallas import tpu as pltpu
from jax.experimental.pallas import tpu_sc as plsc

VOCAB = 1000000
EMB_D = 32
SEQ = 50
NUM_WORKERS = 32
TOK_ROWS = 128
CHUNK = 16
N_CHUNKS = TOK_ROWS // CHUNK

FOLD = 128 // EMB_D
TBLK = 4096
WBLK = TBLK // FOLD
W_ROWS = VOCAB // FOLD


def _fold_table_body(t_ref, w_ref):
    x = t_ref[...].T.reshape(WBLK, FOLD, EMB_D)
    for j in range(FOLD):
        w_ref[:, j * EMB_D:(j + 1) * EMB_D] = x[:, j, :]


def _gather_body(table_hbm, idx_hbm, out_hbm, idx_v, rows_v, sem0, sem1, osem0, osem1):
    wid = lax.axis_index("s") * 2 + lax.axis_index("c")
    base = wid * TOK_ROWS
    pltpu.sync_copy(idx_hbm.at[pl.ds(base, TOK_ROWS)], idx_v)
    gsems = (sem0, sem1)
    osems = (osem0, osem1)
    gd = [None, None]
    od = [None, None]
    for c in range(N_CHUNKS):
        b = c & 1
        if od[b] is not None:
            od[b].wait()
        gd[b] = [
            pltpu.async_copy(
                table_hbm.at[idx_v.at[c * CHUNK + i]], rows_v.at[b, i], gsems[b])
            for i in range(CHUNK)
        ]
        if c > 0:
            pb = (c - 1) & 1
            for dsc in gd[pb]:
                dsc.wait()
            od[pb] = pltpu.async_copy(
                rows_v.at[pb], out_hbm.at[pl.ds(base + (c - 1) * CHUNK, CHUNK)],
                osems[pb])
    lb = (N_CHUNKS - 1) & 1
    for dsc in gd[lb]:
        dsc.wait()
    od[lb] = pltpu.async_copy(
        rows_v.at[lb], out_hbm.at[pl.ds(base + (N_CHUNKS - 1) * CHUNK, CHUNK)],
        osems[lb])
    od[1 - lb].wait()
    od[lb].wait()


@jax.jit
def _embedding_lookup(token_ids, embedding_matrix):
    tab_t = embedding_matrix.T
    n_blk = (VOCAB + TBLK - 1) // TBLK
    w = pl.pallas_call(
        _fold_table_body,
        grid=(n_blk,),
        in_specs=[pl.BlockSpec((EMB_D, TBLK), lambda k: (0, k))],
        out_specs=pl.BlockSpec((WBLK, 128), lambda k: (k, 0)),
        out_shape=jax.ShapeDtypeStruct((W_ROWS, 128), jnp.float32),
    )(tab_t)

    table_rm = w.reshape(VOCAB, EMB_D)
    idx = token_ids.astype(jnp.int32)
    mesh = plsc.VectorSubcoreMesh(core_axis_name="c", subcore_axis_name="s")
    k = functools.partial(
        pl.kernel,
        mesh=mesh,
        out_type=jax.ShapeDtypeStruct((4096, SEQ, EMB_D), jnp.float32),
        scratch_types=[
            pltpu.VMEM((TOK_ROWS, SEQ), jnp.int32),
            pltpu.VMEM((2, CHUNK, SEQ, EMB_D), jnp.float32),
            pltpu.SemaphoreType.DMA,
            pltpu.SemaphoreType.DMA,
            pltpu.SemaphoreType.DMA,
            pltpu.SemaphoreType.DMA,
        ],
        compiler_params=pltpu.CompilerParams(use_tc_tiling_on_sc=False),
    )(_gather_body)
    return k(table_rm, idx)


def kernel(token_ids, embedding_matrix):
    return _embedding_lookup(token_ids, embedding_matrix)


# speedup vs baseline: 1.6902x; 1.1945x over previous
import functools

import jax
import jax.numpy as jnp
from jax import lax
from jax.experimental import pallas as pl
from jax.experimental.pallas import tpu as pltpu
from jax.experimental.pallas import tpu_sc as plsc

VOCAB = 1000000
EMB_D = 32
SEQ = 50
NUM_WORKERS = 32
TOK_COLS = 128
CHUNK = 10
N_CHUNKS = SEQ // CHUNK

FOLD = 128 // EMB_D
TBLK = 4096
STRIP = TBLK // FOLD
N_BLK = (VOCAB + TBLK - 1) // TBLK
W_ROWS = N_BLK * STRIP
W_FLAT = W_ROWS * FOLD


def _fold_table_body(t_ref, w_ref):
    for j in range(FOLD):
        w_ref[:, j * EMB_D:(j + 1) * EMB_D] = t_ref[:, j * STRIP:(j + 1) * STRIP].T


def _idx_body(t_ref, o_ref):
    r = t_ref[...]
    g = (r & ~4095) | ((r & 1023) << 2) | ((r >> 10) & 3)
    o_ref[0:SEQ, :] = g
    o_ref[SEQ:56, :] = jnp.zeros((56 - SEQ, 4096), jnp.int32)


def _gather_body(table_hbm, idx_hbm, out_hbm, idx_v, rows_v, sem0, sem1, osem0, osem1):
    wid = lax.axis_index("s") * 2 + lax.axis_index("c")
    base = wid * TOK_COLS
    pltpu.sync_copy(idx_hbm.at[pl.ds(0, SEQ), pl.ds(base, TOK_COLS)], idx_v)
    gsems = (sem0, sem1)
    osems = (osem0, osem1)
    gd = [None, None]
    od = [None, None]
    for c in range(N_CHUNKS):
        b = c & 1
        if od[b] is not None:
            for dsc in od[b]:
                dsc.wait()
        gd[b] = [
            pltpu.async_copy(
                table_hbm.at[idx_v.at[c * CHUNK + i]], rows_v.at[b, i], gsems[b])
            for i in range(CHUNK)
        ]
        if c > 0:
            pb = (c - 1) & 1
            for dsc in gd[pb]:
                dsc.wait()
            od[pb] = [
                pltpu.async_copy(
                    rows_v.at[pb, i],
                    out_hbm.at[pl.ds(base, TOK_COLS), (c - 1) * CHUNK + i],
                    osems[pb])
                for i in range(CHUNK)
            ]
    lb = (N_CHUNKS - 1) & 1
    for dsc in gd[lb]:
        dsc.wait()
    od[lb] = [
        pltpu.async_copy(
            rows_v.at[lb, i],
            out_hbm.at[pl.ds(base, TOK_COLS), (N_CHUNKS - 1) * CHUNK + i],
            osems[lb])
        for i in range(CHUNK)
    ]
    for dsc in od[1 - lb]:
        dsc.wait()
    for dsc in od[lb]:
        dsc.wait()


@jax.jit
def _embedding_lookup(token_ids, embedding_matrix):
    tab_t = embedding_matrix.T
    w = pl.pallas_call(
        _fold_table_body,
        grid=(N_BLK,),
        in_specs=[pl.BlockSpec((EMB_D, TBLK), lambda k: (0, k))],
        out_specs=pl.BlockSpec((STRIP, 128), lambda k: (k, 0)),
        out_shape=jax.ShapeDtypeStruct((W_ROWS, 128), jnp.float32),
    )(tab_t)

    tid_t = token_ids.astype(jnp.int32).T
    idx_t = pl.pallas_call(
        _idx_body,
        out_shape=jax.ShapeDtypeStruct((56, 4096), jnp.int32),
    )(tid_t)

    table_rm = w.reshape(W_FLAT, EMB_D)
    mesh = plsc.VectorSubcoreMesh(core_axis_name="c", subcore_axis_name="s")
    k = functools.partial(
        pl.kernel,
        mesh=mesh,
        out_type=jax.ShapeDtypeStruct((4096, SEQ, EMB_D), jnp.float32),
        scratch_types=[
            pltpu.VMEM((SEQ, TOK_COLS), jnp.int32),
            pltpu.VMEM((2, CHUNK, TOK_COLS, EMB_D), jnp.float32),
            pltpu.SemaphoreType.DMA,
            pltpu.SemaphoreType.DMA,
            pltpu.SemaphoreType.DMA,
            pltpu.SemaphoreType.DMA,
        ],
        compiler_params=pltpu.CompilerParams(use_tc_tiling_on_sc=False),
    )(_gather_body)
    return k(table_rm, idx_t)


def kernel(token_ids, embedding_matrix):
    return _embedding_lookup(token_ids, embedding_matrix)


# speedup vs baseline: 2.1628x; 1.2796x over previous
import functools

import jax
import jax.numpy as jnp
from jax import lax
from jax.experimental import pallas as pl
from jax.experimental.pallas import tpu as pltpu
from jax.experimental.pallas import tpu_sc as plsc

VOCAB = 1000000
EMB_D = 32
SEQ = 50
NUM_WORKERS = 32
TOK_COLS = 128
CHUNK = 10
N_CHUNKS = SEQ // CHUNK

FOLD = 128 // EMB_D
TBLK = 4096
STRIP = TBLK // FOLD
N_BLK = (VOCAB + TBLK - 1) // TBLK
W_ROWS = N_BLK * STRIP
W_FLAT = W_ROWS * FOLD


def _fold_table_body(t_ref, w_ref):
    t = t_ref[...]
    t_r = jnp.concatenate(
        [t[:, j * STRIP:(j + 1) * STRIP] for j in range(FOLD)], axis=0)
    w_ref[...] = t_r.T


def _idx_body(t_ref, o_ref):
    r = t_ref[...]
    g = (r & ~4095) | ((r & 1023) << 2) | ((r >> 10) & 3)
    o_ref[0:SEQ, :] = g
    o_ref[SEQ:56, :] = jnp.zeros((56 - SEQ, 4096), jnp.int32)


def _gather_body(table_hbm, idx_hbm, out_hbm, idx_v, rows_v, sem0, sem1, osem0, osem1):
    wid = lax.axis_index("s") * 2 + lax.axis_index("c")
    base = wid * TOK_COLS
    pltpu.sync_copy(idx_hbm.at[pl.ds(0, SEQ), pl.ds(base, TOK_COLS)], idx_v)
    gsems = (sem0, sem1)
    osems = (osem0, osem1)
    gd = [None, None]
    od = [None, None]
    for c in range(N_CHUNKS):
        b = c & 1
        if od[b] is not None:
            for dsc in od[b]:
                dsc.wait()
        gd[b] = [
            pltpu.async_copy(
                table_hbm.at[idx_v.at[c * CHUNK + i]], rows_v.at[b, i], gsems[b])
            for i in range(CHUNK)
        ]
        if c > 0:
            pb = (c - 1) & 1
            for dsc in gd[pb]:
                dsc.wait()
            od[pb] = [
                pltpu.async_copy(
                    rows_v.at[pb, i],
                    out_hbm.at[pl.ds(base, TOK_COLS), (c - 1) * CHUNK + i],
                    osems[pb])
                for i in range(CHUNK)
            ]
    lb = (N_CHUNKS - 1) & 1
    for dsc in gd[lb]:
        dsc.wait()
    od[lb] = [
        pltpu.async_copy(
            rows_v.at[lb, i],
            out_hbm.at[pl.ds(base, TOK_COLS), (N_CHUNKS - 1) * CHUNK + i],
            osems[lb])
        for i in range(CHUNK)
    ]
    for dsc in od[1 - lb]:
        dsc.wait()
    for dsc in od[lb]:
        dsc.wait()


@jax.jit
def _embedding_lookup(token_ids, embedding_matrix):
    tab_t = embedding_matrix.T
    w = pl.pallas_call(
        _fold_table_body,
        grid=(N_BLK,),
        in_specs=[pl.BlockSpec((EMB_D, TBLK), lambda k: (0, k))],
        out_specs=pl.BlockSpec((STRIP, 128), lambda k: (k, 0)),
        out_shape=jax.ShapeDtypeStruct((W_ROWS, 128), jnp.float32),
    )(tab_t)

    tid_t = token_ids.astype(jnp.int32).T
    idx_t = pl.pallas_call(
        _idx_body,
        out_shape=jax.ShapeDtypeStruct((56, 4096), jnp.int32),
    )(tid_t)

    table_rm = w.reshape(W_FLAT, EMB_D)
    mesh = plsc.VectorSubcoreMesh(core_axis_name="c", subcore_axis_name="s")
    k = functools.partial(
        pl.kernel,
        mesh=mesh,
        out_type=jax.ShapeDtypeStruct((4096, SEQ, EMB_D), jnp.float32),
        scratch_types=[
            pltpu.VMEM((SEQ, TOK_COLS), jnp.int32),
            pltpu.VMEM((2, CHUNK, TOK_COLS, EMB_D), jnp.float32),
            pltpu.SemaphoreType.DMA,
            pltpu.SemaphoreType.DMA,
            pltpu.SemaphoreType.DMA,
            pltpu.SemaphoreType.DMA,
        ],
        compiler_params=pltpu.CompilerParams(use_tc_tiling_on_sc=False),
    )(_gather_body)
    return k(table_rm, idx_t)


def kernel(token_ids, embedding_matrix):
    return _embedding_lookup(token_ids, embedding_matrix)


# speedup vs baseline: 2.5836x; 1.1945x over previous
import functools

import jax
import jax.numpy as jnp
from jax import lax
from jax.experimental import pallas as pl
from jax.experimental.pallas import tpu as pltpu
from jax.experimental.pallas import tpu_sc as plsc

VOCAB = 1000000
EMB_D = 32
SEQ = 50
NUM_WORKERS = 32
TOK_COLS = 128
CHUNK = 10
N_CHUNKS = SEQ // CHUNK

FOLD = 128 // EMB_D
TBLK = 8192
STRIP = TBLK // FOLD
SHIFT = STRIP.bit_length() - 1
N_BLK = (VOCAB + TBLK - 1) // TBLK
W_ROWS = N_BLK * STRIP
W_FLAT = W_ROWS * FOLD


def _fold_table_body(t_ref, w_ref):
    t = t_ref[...]
    t_r = jnp.concatenate(
        [t[:, j * STRIP:(j + 1) * STRIP] for j in range(FOLD)], axis=0)
    w_ref[...] = t_r.T


def _idx_body(t_ref, o_ref):
    r = t_ref[...]
    g = (r & ~(TBLK - 1)) | ((r & (STRIP - 1)) << 2) | ((r >> SHIFT) & 3)
    o_ref[0:SEQ, :] = g
    o_ref[SEQ:56, :] = jnp.zeros((56 - SEQ, 4096), jnp.int32)


def _gather_body(table_hbm, idx_hbm, out_hbm, idx_v, rows_v, sem0, sem1, osem0, osem1):
    wid = lax.axis_index("s") * 2 + lax.axis_index("c")
    base = wid * TOK_COLS
    pltpu.sync_copy(idx_hbm.at[pl.ds(0, SEQ), pl.ds(base, TOK_COLS)], idx_v)
    gsems = (sem0, sem1)
    osems = (osem0, osem1)
    gd = [None, None]
    od = [None, None]
    for c in range(N_CHUNKS):
        b = c & 1
        if od[b] is not None:
            for dsc in od[b]:
                dsc.wait()
        gd[b] = [
            pltpu.async_copy(
                table_hbm.at[idx_v.at[c * CHUNK + i]], rows_v.at[b, i], gsems[b])
            for i in range(CHUNK)
        ]
        if c > 0:
            pb = (c - 1) & 1
            for dsc in gd[pb]:
                dsc.wait()
            od[pb] = [
                pltpu.async_copy(
                    rows_v.at[pb, i],
                    out_hbm.at[pl.ds(base, TOK_COLS), (c - 1) * CHUNK + i],
                    osems[pb])
                for i in range(CHUNK)
            ]
    lb = (N_CHUNKS - 1) & 1
    for dsc in gd[lb]:
        dsc.wait()
    od[lb] = [
        pltpu.async_copy(
            rows_v.at[lb, i],
            out_hbm.at[pl.ds(base, TOK_COLS), (N_CHUNKS - 1) * CHUNK + i],
            osems[lb])
        for i in range(CHUNK)
    ]
    for dsc in od[1 - lb]:
        dsc.wait()
    for dsc in od[lb]:
        dsc.wait()


@jax.jit
def _embedding_lookup(token_ids, embedding_matrix):
    tab_t = embedding_matrix.T
    w = pl.pallas_call(
        _fold_table_body,
        grid=(N_BLK,),
        in_specs=[pl.BlockSpec((EMB_D, TBLK), lambda k: (0, k))],
        out_specs=pl.BlockSpec((STRIP, 128), lambda k: (k, 0)),
        out_shape=jax.ShapeDtypeStruct((W_ROWS, 128), jnp.float32),
    )(tab_t)

    tid_t = token_ids.astype(jnp.int32).T
    idx_t = pl.pallas_call(
        _idx_body,
        out_shape=jax.ShapeDtypeStruct((56, 4096), jnp.int32),
    )(tid_t)

    table_rm = w.reshape(W_FLAT, EMB_D)
    mesh = plsc.VectorSubcoreMesh(core_axis_name="c", subcore_axis_name="s")
    k = functools.partial(
        pl.kernel,
        mesh=mesh,
        out_type=jax.ShapeDtypeStruct((4096, SEQ, EMB_D), jnp.float32),
        scratch_types=[
            pltpu.VMEM((SEQ, TOK_COLS), jnp.int32),
            pltpu.VMEM((2, CHUNK, TOK_COLS, EMB_D), jnp.float32),
            pltpu.SemaphoreType.DMA,
            pltpu.SemaphoreType.DMA,
            pltpu.SemaphoreType.DMA,
            pltpu.SemaphoreType.DMA,
        ],
        compiler_params=pltpu.CompilerParams(use_tc_tiling_on_sc=False),
    )(_gather_body)
    return k(table_rm, idx_t)


def kernel(token_ids, embedding_matrix):
    return _embedding_lookup(token_ids, embedding_matrix)


# speedup vs baseline: 2.9599x; 1.1457x over previous
import functools

import jax
import jax.numpy as jnp
from jax import lax
from jax.experimental import pallas as pl
from jax.experimental.pallas import tpu as pltpu
from jax.experimental.pallas import tpu_sc as plsc

VOCAB = 1000000
EMB_D = 32
SEQ = 50
NUM_WORKERS = 32
TOK_COLS = 128
CHUNK = 10
N_CHUNKS = SEQ // CHUNK

FOLD = 128 // EMB_D
TBLK = 16384
STRIP = TBLK // FOLD
SHIFT = STRIP.bit_length() - 1
N_BLK = (VOCAB + TBLK - 1) // TBLK
W_ROWS = N_BLK * STRIP
W_FLAT = W_ROWS * FOLD


def _fold_table_body(t_ref, w_ref):
    t = t_ref[...]
    t_r = jnp.concatenate(
        [t[:, j * STRIP:(j + 1) * STRIP] for j in range(FOLD)], axis=0)
    w_ref[...] = t_r.T


def _idx_body(t_ref, o_ref):
    r = t_ref[...]
    g = (r & ~(TBLK - 1)) | ((r & (STRIP - 1)) << 2) | ((r >> SHIFT) & 3)
    o_ref[0:SEQ, :] = g
    o_ref[SEQ:56, :] = jnp.zeros((56 - SEQ, 4096), jnp.int32)


def _gather_body(table_hbm, idx_hbm, out_hbm, idx_v, rows_v, sem0, sem1, osem0, osem1):
    wid = lax.axis_index("s") * 2 + lax.axis_index("c")
    base = wid * TOK_COLS
    pltpu.sync_copy(idx_hbm.at[pl.ds(0, SEQ), pl.ds(base, TOK_COLS)], idx_v)
    gsems = (sem0, sem1)
    osems = (osem0, osem1)
    gd = [None, None]
    od = [None, None]
    for c in range(N_CHUNKS):
        b = c & 1
        if od[b] is not None:
            for dsc in od[b]:
                dsc.wait()
        gd[b] = [
            pltpu.async_copy(
                table_hbm.at[idx_v.at[c * CHUNK + i]], rows_v.at[b, i], gsems[b])
            for i in range(CHUNK)
        ]
        if c > 0:
            pb = (c - 1) & 1
            for dsc in gd[pb]:
                dsc.wait()
            od[pb] = [
                pltpu.async_copy(
                    rows_v.at[pb, i],
                    out_hbm.at[pl.ds(base, TOK_COLS), (c - 1) * CHUNK + i],
                    osems[pb])
                for i in range(CHUNK)
            ]
    lb = (N_CHUNKS - 1) & 1
    for dsc in gd[lb]:
        dsc.wait()
    od[lb] = [
        pltpu.async_copy(
            rows_v.at[lb, i],
            out_hbm.at[pl.ds(base, TOK_COLS), (N_CHUNKS - 1) * CHUNK + i],
            osems[lb])
        for i in range(CHUNK)
    ]
    for dsc in od[1 - lb]:
        dsc.wait()
    for dsc in od[lb]:
        dsc.wait()


@jax.jit
def _embedding_lookup(token_ids, embedding_matrix):
    tab_t = embedding_matrix.T
    w = pl.pallas_call(
        _fold_table_body,
        grid=(N_BLK,),
        in_specs=[pl.BlockSpec((EMB_D, TBLK), lambda k: (0, k))],
        out_specs=pl.BlockSpec((STRIP, 128), lambda k: (k, 0)),
        out_shape=jax.ShapeDtypeStruct((W_ROWS, 128), jnp.float32),
    )(tab_t)

    tid_t = token_ids.astype(jnp.int32).T
    idx_t = pl.pallas_call(
        _idx_body,
        out_shape=jax.ShapeDtypeStruct((56, 4096), jnp.int32),
    )(tid_t)

    table_rm = w.reshape(W_FLAT, EMB_D)
    mesh = plsc.VectorSubcoreMesh(core_axis_name="c", subcore_axis_name="s")
    k = functools.partial(
        pl.kernel,
        mesh=mesh,
        out_type=jax.ShapeDtypeStruct((4096, SEQ, EMB_D), jnp.float32),
        scratch_types=[
            pltpu.VMEM((SEQ, TOK_COLS), jnp.int32),
            pltpu.VMEM((2, CHUNK, TOK_COLS, EMB_D), jnp.float32),
            pltpu.SemaphoreType.DMA,
            pltpu.SemaphoreType.DMA,
            pltpu.SemaphoreType.DMA,
            pltpu.SemaphoreType.DMA,
        ],
        compiler_params=pltpu.CompilerParams(use_tc_tiling_on_sc=False),
    )(_gather_body)
    return k(table_rm, idx_t)


def kernel(token_ids, embedding_matrix):
    return _embedding_lookup(token_ids, embedding_matrix)


# speedup vs baseline: 3.1335x; 1.0587x over previous
import functools

import jax
import jax.numpy as jnp
from jax import lax
from jax.experimental import pallas as pl
from jax.experimental.pallas import tpu as pltpu
from jax.experimental.pallas import tpu_sc as plsc

VOCAB = 1000000
EMB_D = 32
SEQ = 50
NUM_WORKERS = 32
TOK_COLS = 128
CHUNK = 10
N_CHUNKS = SEQ // CHUNK

FOLD = 128 // EMB_D
TBLK = 32768
STRIP = TBLK // FOLD
SHIFT = STRIP.bit_length() - 1
N_BLK = (VOCAB + TBLK - 1) // TBLK
W_ROWS = N_BLK * STRIP
W_FLAT = W_ROWS * FOLD


def _fold_table_body(t_ref, w_ref):
    t = t_ref[...]
    t_r = jnp.concatenate(
        [t[:, j * STRIP:(j + 1) * STRIP] for j in range(FOLD)], axis=0)
    w_ref[...] = t_r.T


def _idx_body(t_ref, o_ref):
    r = t_ref[...]
    g = (r & ~(TBLK - 1)) | ((r & (STRIP - 1)) << 2) | ((r >> SHIFT) & 3)
    o_ref[0:SEQ, :] = g
    o_ref[SEQ:56, :] = jnp.zeros((56 - SEQ, 4096), jnp.int32)


def _gather_body(table_hbm, idx_hbm, out_hbm, idx_v, rows_v, sem0, sem1, osem0, osem1):
    wid = lax.axis_index("s") * 2 + lax.axis_index("c")
    base = wid * TOK_COLS
    pltpu.sync_copy(idx_hbm.at[pl.ds(0, SEQ), pl.ds(base, TOK_COLS)], idx_v)
    gsems = (sem0, sem1)
    osems = (osem0, osem1)
    gd = [None, None]
    od = [None, None]
    for c in range(N_CHUNKS):
        b = c & 1
        if od[b] is not None:
            for dsc in od[b]:
                dsc.wait()
        gd[b] = [
            pltpu.async_copy(
                table_hbm.at[idx_v.at[c * CHUNK + i]], rows_v.at[b, i], gsems[b])
            for i in range(CHUNK)
        ]
        if c > 0:
            pb = (c - 1) & 1
            for dsc in gd[pb]:
                dsc.wait()
            od[pb] = [
                pltpu.async_copy(
                    rows_v.at[pb, i],
                    out_hbm.at[pl.ds(base, TOK_COLS), (c - 1) * CHUNK + i],
                    osems[pb])
                for i in range(CHUNK)
            ]
    lb = (N_CHUNKS - 1) & 1
    for dsc in gd[lb]:
        dsc.wait()
    od[lb] = [
        pltpu.async_copy(
            rows_v.at[lb, i],
            out_hbm.at[pl.ds(base, TOK_COLS), (N_CHUNKS - 1) * CHUNK + i],
            osems[lb])
        for i in range(CHUNK)
    ]
    for dsc in od[1 - lb]:
        dsc.wait()
    for dsc in od[lb]:
        dsc.wait()


@jax.jit
def _embedding_lookup(token_ids, embedding_matrix):
    tab_t = embedding_matrix.T
    w = pl.pallas_call(
        _fold_table_body,
        grid=(N_BLK,),
        in_specs=[pl.BlockSpec((EMB_D, TBLK), lambda k: (0, k))],
        out_specs=pl.BlockSpec((STRIP, 128), lambda k: (k, 0)),
        out_shape=jax.ShapeDtypeStruct((W_ROWS, 128), jnp.float32),
    )(tab_t)

    tid_t = token_ids.astype(jnp.int32).T
    idx_t = pl.pallas_call(
        _idx_body,
        out_shape=jax.ShapeDtypeStruct((56, 4096), jnp.int32),
    )(tid_t)

    table_rm = w.reshape(W_FLAT, EMB_D)
    mesh = plsc.VectorSubcoreMesh(core_axis_name="c", subcore_axis_name="s")
    k = functools.partial(
        pl.kernel,
        mesh=mesh,
        out_type=jax.ShapeDtypeStruct((4096, SEQ, EMB_D), jnp.float32),
        scratch_types=[
            pltpu.VMEM((SEQ, TOK_COLS), jnp.int32),
            pltpu.VMEM((2, CHUNK, TOK_COLS, EMB_D), jnp.float32),
            pltpu.SemaphoreType.DMA,
            pltpu.SemaphoreType.DMA,
            pltpu.SemaphoreType.DMA,
            pltpu.SemaphoreType.DMA,
        ],
        compiler_params=pltpu.CompilerParams(use_tc_tiling_on_sc=False),
    )(_gather_body)
    return k(table_rm, idx_t)


def kernel(token_ids, embedding_matrix):
    return _embedding_lookup(token_ids, embedding_matrix)


# speedup vs baseline: 3.1512x; 1.0056x over previous
import functools

import jax
import jax.numpy as jnp
from jax import lax
from jax.experimental import pallas as pl
from jax.experimental.pallas import tpu as pltpu
from jax.experimental.pallas import tpu_sc as plsc

VOCAB = 1000000
EMB_D = 32
SEQ = 50
NUM_WORKERS = 32
TOK_COLS = 128
CHUNK = 10
N_CHUNKS = SEQ // CHUNK

FOLD = 128 // EMB_D
TBLK = 65536
STRIP = TBLK // FOLD
SHIFT = STRIP.bit_length() - 1
N_BLK = (VOCAB + TBLK - 1) // TBLK
W_ROWS = N_BLK * STRIP
W_FLAT = W_ROWS * FOLD


def _fold_table_body(t_ref, w_ref):
    t = t_ref[...]
    t_r = jnp.concatenate(
        [t[:, j * STRIP:(j + 1) * STRIP] for j in range(FOLD)], axis=0)
    w_ref[...] = t_r.T


def _idx_body(t_ref, o_ref):
    r = t_ref[...]
    g = (r & ~(TBLK - 1)) | ((r & (STRIP - 1)) << 2) | ((r >> SHIFT) & 3)
    o_ref[0:SEQ, :] = g
    o_ref[SEQ:56, :] = jnp.zeros((56 - SEQ, 4096), jnp.int32)


def _gather_body(table_hbm, idx_hbm, out_hbm, idx_v, rows_v, sem0, sem1, osem0, osem1):
    wid = lax.axis_index("s") * 2 + lax.axis_index("c")
    base = wid * TOK_COLS
    pltpu.sync_copy(idx_hbm.at[pl.ds(0, SEQ), pl.ds(base, TOK_COLS)], idx_v)
    gsems = (sem0, sem1)
    osems = (osem0, osem1)
    gd = [None, None]
    od = [None, None]
    for c in range(N_CHUNKS):
        b = c & 1
        if od[b] is not None:
            for dsc in od[b]:
                dsc.wait()
        gd[b] = [
            pltpu.async_copy(
                table_hbm.at[idx_v.at[c * CHUNK + i]], rows_v.at[b, i], gsems[b])
            for i in range(CHUNK)
        ]
        if c > 0:
            pb = (c - 1) & 1
            for dsc in gd[pb]:
                dsc.wait()
            od[pb] = [
                pltpu.async_copy(
                    rows_v.at[pb, i],
                    out_hbm.at[pl.ds(base, TOK_COLS), (c - 1) * CHUNK + i],
                    osems[pb])
                for i in range(CHUNK)
            ]
    lb = (N_CHUNKS - 1) & 1
    for dsc in gd[lb]:
        dsc.wait()
    od[lb] = [
        pltpu.async_copy(
            rows_v.at[lb, i],
            out_hbm.at[pl.ds(base, TOK_COLS), (N_CHUNKS - 1) * CHUNK + i],
            osems[lb])
        for i in range(CHUNK)
    ]
    for dsc in od[1 - lb]:
        dsc.wait()
    for dsc in od[lb]:
        dsc.wait()


@jax.jit
def _embedding_lookup(token_ids, embedding_matrix):
    tab_t = embedding_matrix.T
    w = pl.pallas_call(
        _fold_table_body,
        grid=(N_BLK,),
        in_specs=[pl.BlockSpec((EMB_D, TBLK), lambda k: (0, k))],
        out_specs=pl.BlockSpec((STRIP, 128), lambda k: (k, 0)),
        out_shape=jax.ShapeDtypeStruct((W_ROWS, 128), jnp.float32),
    )(tab_t)

    tid_t = token_ids.astype(jnp.int32).T
    idx_t = pl.pallas_call(
        _idx_body,
        out_shape=jax.ShapeDtypeStruct((56, 4096), jnp.int32),
    )(tid_t)

    table_rm = w.reshape(W_FLAT, EMB_D)
    mesh = plsc.VectorSubcoreMesh(core_axis_name="c", subcore_axis_name="s")
    k = functools.partial(
        pl.kernel,
        mesh=mesh,
        out_type=jax.ShapeDtypeStruct((4096, SEQ, EMB_D), jnp.float32),
        scratch_types=[
            pltpu.VMEM((SEQ, TOK_COLS), jnp.int32),
            pltpu.VMEM((2, CHUNK, TOK_COLS, EMB_D), jnp.float32),
            pltpu.SemaphoreType.DMA,
            pltpu.SemaphoreType.DMA,
            pltpu.SemaphoreType.DMA,
            pltpu.SemaphoreType.DMA,
        ],
        compiler_params=pltpu.CompilerParams(use_tc_tiling_on_sc=False),
    )(_gather_body)
    return k(table_rm, idx_t)


def kernel(token_ids, embedding_matrix):
    return _embedding_lookup(token_ids, embedding_matrix)
